# Initial kernel scaffold; baseline (speedup 1.0000x reference)
import jax
import jax.numpy as jnp
from jax.experimental import pallas as pl


def kernel(x, adj, W1, b1, W2, b2, gamma1, beta1, gamma2, beta2):
    raise NotImplementedError("write your pallas kernel here")


# speedup vs baseline: 1.0102x; 1.0102x over previous
import functools

import jax
import jax.numpy as jnp
from jax.experimental import pallas as pl

EPS = 1e-5


def _finalize(s_ref, q_ref, g_ref, be_ref, count):
    inv = 1.0 / count
    tot = jnp.sum(s_ref[...], axis=(0, 1))
    tot2 = jnp.sum(q_ref[...], axis=(0, 1))
    mean = tot * inv
    var = tot2 * inv - mean * mean
    a = g_ref[0] * jax.lax.rsqrt(var + EPS)
    c = be_ref[0] - mean * a
    return a, c


def _k1(x_ref, adj_ref, w_ref, b_ref, y_ref, s_ref, q_ref):
    s = jnp.dot(x_ref[0], w_ref[...], preferred_element_type=jnp.float32)
    y = jnp.dot(adj_ref[0], s, preferred_element_type=jnp.float32) + b_ref[...]
    y = jnp.maximum(y, 0.0)
    y_ref[0] = y
    n = y.shape[0]
    s_ref[...] = jnp.sum(y, axis=1).reshape(1, 1, n)
    q_ref[...] = jnp.sum(y * y, axis=1).reshape(1, 1, n)


def _k2(y1_ref, adj_ref, w_ref, b_ref, s_ref, q_ref, g_ref, be_ref,
        y2_ref, s2_ref, q2_ref, *, count):
    a, c = _finalize(s_ref, q_ref, g_ref, be_ref, count)
    h = y1_ref[0] * a[:, None] + c[:, None]
    s2 = jnp.dot(h, w_ref[...], preferred_element_type=jnp.float32)
    y2 = jnp.dot(adj_ref[0], s2, preferred_element_type=jnp.float32) + b_ref[...]
    y2 = jnp.maximum(y2, 0.0)
    y2_ref[0] = y2
    n = y2.shape[0]
    s2_ref[...] = jnp.sum(y2, axis=1).reshape(1, 1, n)
    q2_ref[...] = jnp.sum(y2 * y2, axis=1).reshape(1, 1, n)


def _k3(y2_ref, s_ref, q_ref, g_ref, be_ref, out_ref, *, count):
    a, c = _finalize(s_ref, q_ref, g_ref, be_ref, count)
    out_ref[0] = y2_ref[0] * a[:, None] + c[:, None]


@jax.jit
def kernel(x, adj, W1, b1, W2, b2, gamma1, beta1, gamma2, beta2):
    B, N, C_in = x.shape
    C_hid = W1.shape[1]
    C_out = W2.shape[1]
    f32 = jnp.float32

    full = lambda shape: pl.BlockSpec(shape, lambda b: (0,) * len(shape))
    per_b = lambda *dims: pl.BlockSpec((1,) + dims, lambda b: (b,) + (0,) * len(dims))

    y1, s1, q1 = pl.pallas_call(
        _k1,
        grid=(B,),
        in_specs=[per_b(N, C_in), per_b(N, N), full((C_in, C_hid)),
                  full((1, C_hid))],
        out_specs=[per_b(N, C_hid), per_b(1, N), per_b(1, N)],
        out_shape=[jax.ShapeDtypeStruct((B, N, C_hid), f32),
                   jax.ShapeDtypeStruct((B, 1, N), f32),
                   jax.ShapeDtypeStruct((B, 1, N), f32)],
    )(x, adj, W1, b1.reshape(1, C_hid))

    y2, s2, q2 = pl.pallas_call(
        functools.partial(_k2, count=B * C_hid),
        grid=(B,),
        in_specs=[per_b(N, C_hid), per_b(N, N), full((C_hid, C_out)),
                  full((1, C_out)), full((B, 1, N)), full((B, 1, N)),
                  full((1, N)), full((1, N))],
        out_specs=[per_b(N, C_out), per_b(1, N), per_b(1, N)],
        out_shape=[jax.ShapeDtypeStruct((B, N, C_out), f32),
                   jax.ShapeDtypeStruct((B, 1, N), f32),
                   jax.ShapeDtypeStruct((B, 1, N), f32)],
    )(y1, adj, W2, b2.reshape(1, C_out), s1, q1,
      gamma1.reshape(1, N), beta1.reshape(1, N))

    out = pl.pallas_call(
        functools.partial(_k3, count=B * C_out),
        grid=(B,),
        in_specs=[per_b(N, C_out), full((B, 1, N)), full((B, 1, N)),
                  full((1, N)), full((1, N))],
        out_specs=per_b(N, C_out),
        out_shape=jax.ShapeDtypeStruct((B, N, C_out), f32),
    )(y2, s2, q2, gamma2.reshape(1, N), beta2.reshape(1, N))

    return out


# speedup vs baseline: 1.2075x; 1.1953x over previous
import functools

import jax
import jax.numpy as jnp
from jax.experimental import pallas as pl

EPS = 1e-5


def _k1(x_ref, adj_ref, w_ref, b_ref, y_ref, s_ref, q_ref):
    s = jnp.dot(x_ref[0], w_ref[...], preferred_element_type=jnp.float32)
    y = jnp.dot(adj_ref[0], s, preferred_element_type=jnp.float32) + b_ref[...]
    y = jnp.maximum(y, 0.0)
    y_ref[0] = y
    s_ref[0] = jnp.sum(y, axis=1, keepdims=True)
    q_ref[0] = jnp.sum(y * y, axis=1, keepdims=True)


def _fin(s_ref, q_ref, g_ref, be_ref, a_ref, c_ref, *, count):
    inv = 1.0 / count
    mean = jnp.sum(s_ref[...], axis=0) * inv
    var = jnp.sum(q_ref[...], axis=0) * inv - mean * mean
    a = g_ref[...] * jax.lax.rsqrt(var + EPS)
    a_ref[...] = a
    c_ref[...] = be_ref[...] - mean * a


def _k2(y1_ref, adj_ref, w_ref, b_ref, a_ref, c_ref, y2_ref, s_ref, q_ref):
    h = y1_ref[0] * a_ref[...] + c_ref[...]
    s2 = jnp.dot(h, w_ref[...], preferred_element_type=jnp.float32)
    y2 = jnp.dot(adj_ref[0], s2, preferred_element_type=jnp.float32) + b_ref[...]
    y2 = jnp.maximum(y2, 0.0)
    y2_ref[0] = y2
    s_ref[0] = jnp.sum(y2, axis=1, keepdims=True)
    q_ref[0] = jnp.sum(y2 * y2, axis=1, keepdims=True)


def _k3(y2_ref, a_ref, c_ref, out_ref):
    out_ref[0] = y2_ref[0] * a_ref[...] + c_ref[...]


@jax.jit
def kernel(x, adj, W1, b1, W2, b2, gamma1, beta1, gamma2, beta2):
    B, N, C_in = x.shape
    C_hid = W1.shape[1]
    C_out = W2.shape[1]
    f32 = jnp.float32

    full = lambda shape: pl.BlockSpec(shape, lambda b: (0,) * len(shape))
    per_b = lambda *dims: pl.BlockSpec((1,) + dims, lambda b: (b,) + (0,) * len(dims))
    stat = jax.ShapeDtypeStruct((B, N, 1), f32)
    vec = jax.ShapeDtypeStruct((N, 1), f32)

    def finalize(s, q, g, be, count):
        return pl.pallas_call(
            functools.partial(_fin, count=count),
            grid=(1,),
            in_specs=[full((B, N, 1)), full((B, N, 1)), full((N, 1)),
                      full((N, 1))],
            out_specs=[full((N, 1)), full((N, 1))],
            out_shape=[vec, vec],
        )(s, q, g.reshape(N, 1), be.reshape(N, 1))

    y1, s1, q1 = pl.pallas_call(
        _k1,
        grid=(B,),
        in_specs=[per_b(N, C_in), per_b(N, N), full((C_in, C_hid)),
                  full((1, C_hid))],
        out_specs=[per_b(N, C_hid), per_b(N, 1), per_b(N, 1)],
        out_shape=[jax.ShapeDtypeStruct((B, N, C_hid), f32), stat, stat],
    )(x, adj, W1, b1.reshape(1, C_hid))

    a1, c1 = finalize(s1, q1, gamma1, beta1, B * C_hid)

    y2, s2, q2 = pl.pallas_call(
        _k2,
        grid=(B,),
        in_specs=[per_b(N, C_hid), per_b(N, N), full((C_hid, C_out)),
                  full((1, C_out)), full((N, 1)), full((N, 1))],
        out_specs=[per_b(N, C_out), per_b(N, 1), per_b(N, 1)],
        out_shape=[jax.ShapeDtypeStruct((B, N, C_out), f32), stat, stat],
    )(y1, adj, W2, b2.reshape(1, C_out), a1, c1)

    a2, c2 = finalize(s2, q2, gamma2, beta2, B * C_out)

    out = pl.pallas_call(
        _k3,
        grid=(B,),
        in_specs=[per_b(N, C_out), full((N, 1)), full((N, 1))],
        out_specs=per_b(N, C_out),
        out_shape=jax.ShapeDtypeStruct((B, N, C_out), f32),
    )(y2, a2, c2)

    return out


# speedup vs baseline: 1.2991x; 1.0758x over previous
import functools

import jax
import jax.numpy as jnp
from jax.experimental import pallas as pl

EPS = 1e-5


def _k1(x_ref, adj_ref, w_ref, b_ref, y_ref, s_ref, q_ref):
    s = jnp.dot(x_ref[0], w_ref[...], preferred_element_type=jnp.float32)
    y = jnp.dot(adj_ref[0], s, preferred_element_type=jnp.float32) + b_ref[...]
    y = jnp.maximum(y, 0.0)
    y_ref[0] = y.astype(y_ref.dtype)
    s_ref[0] = jnp.sum(y, axis=1, keepdims=True)
    q_ref[0] = jnp.sum(y * y, axis=1, keepdims=True)


def _fin(s_ref, q_ref, g_ref, be_ref, a_ref, c_ref, *, count):
    inv = 1.0 / count
    mean = jnp.sum(s_ref[...], axis=0) * inv
    var = jnp.sum(q_ref[...], axis=0) * inv - mean * mean
    a = g_ref[...] * jax.lax.rsqrt(var + EPS)
    a_ref[...] = a
    c_ref[...] = be_ref[...] - mean * a


def _k2(y1_ref, adj_ref, w_ref, b_ref, a_ref, c_ref, y2_ref, s_ref, q_ref):
    h = y1_ref[0].astype(jnp.float32) * a_ref[...] + c_ref[...]
    s2 = jnp.dot(h, w_ref[...], preferred_element_type=jnp.float32)
    y2 = jnp.dot(adj_ref[0], s2, preferred_element_type=jnp.float32) + b_ref[...]
    y2 = jnp.maximum(y2, 0.0)
    y2_ref[0] = y2.astype(y2_ref.dtype)
    s_ref[0] = jnp.sum(y2, axis=1, keepdims=True)
    q_ref[0] = jnp.sum(y2 * y2, axis=1, keepdims=True)


def _k3(y2_ref, a_ref, c_ref, out_ref):
    out_ref[0] = y2_ref[0].astype(jnp.float32) * a_ref[...] + c_ref[...]


@jax.jit
def kernel(x, adj, W1, b1, W2, b2, gamma1, beta1, gamma2, beta2):
    B, N, C_in = x.shape
    C_hid = W1.shape[1]
    C_out = W2.shape[1]
    f32 = jnp.float32

    full = lambda shape: pl.BlockSpec(shape, lambda b: (0,) * len(shape))
    per_b = lambda *dims: pl.BlockSpec((1,) + dims, lambda b: (b,) + (0,) * len(dims))
    stat = jax.ShapeDtypeStruct((B, N, 1), f32)
    vec = jax.ShapeDtypeStruct((N, 1), f32)

    def finalize(s, q, g, be, count):
        return pl.pallas_call(
            functools.partial(_fin, count=count),
            grid=(1,),
            in_specs=[full((B, N, 1)), full((B, N, 1)), full((N, 1)),
                      full((N, 1))],
            out_specs=[full((N, 1)), full((N, 1))],
            out_shape=[vec, vec],
        )(s, q, g.reshape(N, 1), be.reshape(N, 1))

    y1, s1, q1 = pl.pallas_call(
        _k1,
        grid=(B,),
        in_specs=[per_b(N, C_in), per_b(N, N), full((C_in, C_hid)),
                  full((1, C_hid))],
        out_specs=[per_b(N, C_hid), per_b(N, 1), per_b(N, 1)],
        out_shape=[jax.ShapeDtypeStruct((B, N, C_hid), jnp.bfloat16),
                   stat, stat],
    )(x, adj, W1, b1.reshape(1, C_hid))

    a1, c1 = finalize(s1, q1, gamma1, beta1, B * C_hid)

    y2, s2, q2 = pl.pallas_call(
        _k2,
        grid=(B,),
        in_specs=[per_b(N, C_hid), per_b(N, N), full((C_hid, C_out)),
                  full((1, C_out)), full((N, 1)), full((N, 1))],
        out_specs=[per_b(N, C_out), per_b(N, 1), per_b(N, 1)],
        out_shape=[jax.ShapeDtypeStruct((B, N, C_out), jnp.bfloat16),
                   stat, stat],
    )(y1, adj, W2, b2.reshape(1, C_out), a1, c1)

    a2, c2 = finalize(s2, q2, gamma2, beta2, B * C_out)

    out = pl.pallas_call(
        _k3,
        grid=(B,),
        in_specs=[per_b(N, C_out), full((N, 1)), full((N, 1))],
        out_specs=per_b(N, C_out),
        out_shape=jax.ShapeDtypeStruct((B, N, C_out), f32),
    )(y2, a2, c2)

    return out


# speedup vs baseline: 1.4351x; 1.1047x over previous
import functools

import jax
import jax.numpy as jnp
from jax.experimental import pallas as pl
from jax.experimental.pallas import tpu as pltpu

EPS = 1e-5


def _k1(x_ref, adj_ref, w_ref, b_ref, y_ref, s_ref, q_ref):
    s = jnp.dot(x_ref[0], w_ref[...], preferred_element_type=jnp.float32)
    y = jnp.dot(adj_ref[0], s, preferred_element_type=jnp.float32) + b_ref[...]
    y = jnp.maximum(y, 0.0)
    y_ref[0] = y.astype(y_ref.dtype)
    s_ref[0] = jnp.sum(y, axis=1, keepdims=True)
    q_ref[0] = jnp.sum(y * y, axis=1, keepdims=True)


def _finalize(s_ref, q_ref, g_ref, be_ref, a_ref, c_ref, count):
    inv = 1.0 / count
    mean = jnp.sum(s_ref[...], axis=0) * inv
    var = jnp.sum(q_ref[...], axis=0) * inv - mean * mean
    a = g_ref[...] * jax.lax.rsqrt(var + EPS)
    a_ref[...] = a
    c_ref[...] = be_ref[...] - mean * a


def _k2(y1_ref, adj_ref, w_ref, b_ref, s1_ref, q1_ref, g_ref, be_ref,
        y2_ref, s_ref, q_ref, a_s, c_s, *, count):
    @pl.when(pl.program_id(0) == 0)
    def _():
        _finalize(s1_ref, q1_ref, g_ref, be_ref, a_s, c_s, count)

    h = y1_ref[0].astype(jnp.float32) * a_s[...] + c_s[...]
    s2 = jnp.dot(h, w_ref[...], preferred_element_type=jnp.float32)
    y2 = jnp.dot(adj_ref[0], s2, preferred_element_type=jnp.float32) + b_ref[...]
    y2 = jnp.maximum(y2, 0.0)
    y2_ref[0] = y2.astype(y2_ref.dtype)
    s_ref[0] = jnp.sum(y2, axis=1, keepdims=True)
    q_ref[0] = jnp.sum(y2 * y2, axis=1, keepdims=True)


def _k3(y2_ref, s2_ref, q2_ref, g_ref, be_ref, out_ref, a_s, c_s, *, count):
    @pl.when(pl.program_id(0) == 0)
    def _():
        _finalize(s2_ref, q2_ref, g_ref, be_ref, a_s, c_s, count)

    out_ref[...] = (y2_ref[...].astype(jnp.float32) * a_s[...][None]
                    + c_s[...][None])


@jax.jit
def kernel(x, adj, W1, b1, W2, b2, gamma1, beta1, gamma2, beta2):
    B, N, C_in = x.shape
    C_hid = W1.shape[1]
    C_out = W2.shape[1]
    f32 = jnp.float32
    bf16 = jnp.bfloat16

    full = lambda shape: pl.BlockSpec(shape, lambda b: (0,) * len(shape))
    per_b = lambda *dims: pl.BlockSpec((1,) + dims, lambda b: (b,) + (0,) * len(dims))
    stat = jax.ShapeDtypeStruct((B, N, 1), f32)
    vec_scratch = pltpu.VMEM((N, 1), f32)

    y1, s1, q1 = pl.pallas_call(
        _k1,
        grid=(B,),
        in_specs=[per_b(N, C_in), per_b(N, N), full((C_in, C_hid)),
                  full((1, C_hid))],
        out_specs=[per_b(N, C_hid), per_b(N, 1), per_b(N, 1)],
        out_shape=[jax.ShapeDtypeStruct((B, N, C_hid), bf16), stat, stat],
    )(x, adj, W1, b1.reshape(1, C_hid))

    y2, s2, q2 = pl.pallas_call(
        functools.partial(_k2, count=B * C_hid),
        grid=(B,),
        in_specs=[per_b(N, C_hid), per_b(N, N), full((C_hid, C_out)),
                  full((1, C_out)), full((B, N, 1)), full((B, N, 1)),
                  full((N, 1)), full((N, 1))],
        out_specs=[per_b(N, C_out), per_b(N, 1), per_b(N, 1)],
        out_shape=[jax.ShapeDtypeStruct((B, N, C_out), bf16), stat, stat],
        scratch_shapes=[vec_scratch, vec_scratch],
    )(y1, adj, W2, b2.reshape(1, C_out), s1, q1,
      gamma1.reshape(N, 1), beta1.reshape(N, 1))

    BB = 4
    out = pl.pallas_call(
        functools.partial(_k3, count=B * C_out),
        grid=(B // BB,),
        in_specs=[pl.BlockSpec((BB, N, C_out), lambda b: (b, 0, 0)),
                  full((B, N, 1)), full((B, N, 1)),
                  full((N, 1)), full((N, 1))],
        out_specs=pl.BlockSpec((BB, N, C_out), lambda b: (b, 0, 0)),
        out_shape=jax.ShapeDtypeStruct((B, N, C_out), f32),
        scratch_shapes=[vec_scratch, vec_scratch],
    )(y2, s2, q2, gamma2.reshape(N, 1), beta2.reshape(N, 1))

    return out


# speedup vs baseline: 1.5303x; 1.0664x over previous
import functools

import jax
import jax.numpy as jnp
from jax.experimental import pallas as pl
from jax.experimental.pallas import tpu as pltpu

EPS = 1e-5


def _k1(x_ref, adj_ref, w_ref, b_ref, y_ref, s_ref, q_ref, *, bb):
    for i in range(bb):
        s = jnp.dot(x_ref[i], w_ref[...], preferred_element_type=jnp.float32)
        y = jnp.dot(adj_ref[i], s, preferred_element_type=jnp.float32)
        y = jnp.maximum(y + b_ref[...], 0.0)
        y_ref[i] = y.astype(y_ref.dtype)
        s_ref[i] = jnp.sum(y, axis=1, keepdims=True)
        q_ref[i] = jnp.sum(y * y, axis=1, keepdims=True)


def _finalize(s_ref, q_ref, g_ref, be_ref, a_ref, c_ref, count):
    inv = 1.0 / count
    mean = jnp.sum(s_ref[...], axis=0) * inv
    var = jnp.sum(q_ref[...], axis=0) * inv - mean * mean
    a = g_ref[...] * jax.lax.rsqrt(var + EPS)
    a_ref[...] = a
    c_ref[...] = be_ref[...] - mean * a


def _k2(y1_ref, adj_ref, w_ref, b_ref, s1_ref, q1_ref, g_ref, be_ref,
        y2_ref, s_ref, q_ref, a_s, c_s, *, count, bb):
    @pl.when(pl.program_id(0) == 0)
    def _():
        _finalize(s1_ref, q1_ref, g_ref, be_ref, a_s, c_s, count)

    for i in range(bb):
        h = y1_ref[i].astype(jnp.float32) * a_s[...] + c_s[...]
        s2 = jnp.dot(h, w_ref[...], preferred_element_type=jnp.float32)
        y2 = jnp.dot(adj_ref[i], s2, preferred_element_type=jnp.float32)
        y2 = jnp.maximum(y2 + b_ref[...], 0.0)
        y2_ref[i] = y2.astype(y2_ref.dtype)
        s_ref[i] = jnp.sum(y2, axis=1, keepdims=True)
        q_ref[i] = jnp.sum(y2 * y2, axis=1, keepdims=True)


def _k3(y2_ref, s2_ref, q2_ref, g_ref, be_ref, out_ref, a_s, c_s, *, count):
    @pl.when(pl.program_id(0) == 0)
    def _():
        _finalize(s2_ref, q2_ref, g_ref, be_ref, a_s, c_s, count)

    out_ref[...] = (y2_ref[...].astype(jnp.float32) * a_s[...][None]
                    + c_s[...][None])


@jax.jit
def kernel(x, adj, W1, b1, W2, b2, gamma1, beta1, gamma2, beta2):
    B, N, C_in = x.shape
    C_hid = W1.shape[1]
    C_out = W2.shape[1]
    f32 = jnp.float32
    bf16 = jnp.bfloat16

    full = lambda shape: pl.BlockSpec(shape, lambda b: (0,) * len(shape))
    blk = lambda *dims: pl.BlockSpec(dims, lambda b: (b,) + (0,) * (len(dims) - 1))
    stat = jax.ShapeDtypeStruct((B, N, 1), f32)
    vec_scratch = pltpu.VMEM((N, 1), f32)

    BB = 2
    y1, s1, q1 = pl.pallas_call(
        functools.partial(_k1, bb=BB),
        grid=(B // BB,),
        in_specs=[blk(BB, N, C_in), blk(BB, N, N), full((C_in, C_hid)),
                  full((1, C_hid))],
        out_specs=[blk(BB, N, C_hid), blk(BB, N, 1), blk(BB, N, 1)],
        out_shape=[jax.ShapeDtypeStruct((B, N, C_hid), bf16), stat, stat],
    )(x, adj, W1, b1.reshape(1, C_hid))

    y2, s2, q2 = pl.pallas_call(
        functools.partial(_k2, count=B * C_hid, bb=BB),
        grid=(B // BB,),
        in_specs=[blk(BB, N, C_hid), blk(BB, N, N), full((C_hid, C_out)),
                  full((1, C_out)), full((B, N, 1)), full((B, N, 1)),
                  full((N, 1)), full((N, 1))],
        out_specs=[blk(BB, N, C_out), blk(BB, N, 1), blk(BB, N, 1)],
        out_shape=[jax.ShapeDtypeStruct((B, N, C_out), bf16), stat, stat],
        scratch_shapes=[vec_scratch, vec_scratch],
    )(y1, adj, W2, b2.reshape(1, C_out), s1, q1,
      gamma1.reshape(N, 1), beta1.reshape(N, 1))

    BB3 = 4
    out = pl.pallas_call(
        functools.partial(_k3, count=B * C_out),
        grid=(B // BB3,),
        in_specs=[blk(BB3, N, C_out), full((B, N, 1)), full((B, N, 1)),
                  full((N, 1)), full((N, 1))],
        out_specs=blk(BB3, N, C_out),
        out_shape=jax.ShapeDtypeStruct((B, N, C_out), f32),
        scratch_shapes=[vec_scratch, vec_scratch],
    )(y2, s2, q2, gamma2.reshape(N, 1), beta2.reshape(N, 1))

    return out
